# Initial kernel scaffold; baseline (speedup 1.0000x reference)
import jax
import jax.numpy as jnp
from jax.experimental import pallas as pl


def kernel(x, f):
    raise NotImplementedError("write your pallas kernel here")


# speedup vs baseline: 4226.3671x; 4226.3671x over previous
import jax
import jax.numpy as jnp
from jax.experimental import pallas as pl
from jax.experimental.pallas import tpu as pltpu

_OUT = 299


def _upsample_body(f_ref, x_ref, o_ref):
    si = pl.program_id(0)
    gi = pl.program_id(1)
    tlx = f_ref[si, gi, 0]
    tly = f_ref[si, gi, 1]
    brx = f_ref[si, gi, 2]
    bry = f_ref[si, gi, 3]
    hc = (brx - tlx).astype(jnp.float32)
    wc = (bry - tly).astype(jnp.float32)

    i2 = jax.lax.broadcasted_iota(jnp.int32, (_OUT, _OUT), 0).astype(jnp.float32)
    k2 = jax.lax.broadcasted_iota(jnp.int32, (_OUT, _OUT), 1)
    src_r = jnp.clip((i2 + 0.5) * (hc / _OUT) - 0.5, 0.0, hc - 1.0)
    r0f = jnp.floor(src_r)
    wr = src_r - r0f
    r0 = r0f.astype(jnp.int32) + tlx
    r1 = jnp.minimum(r0 + 1, brx - 1)
    rmat = jnp.where(k2 == r0, 1.0 - wr, 0.0) + jnp.where(k2 == r1, wr, 0.0)

    j2 = jax.lax.broadcasted_iota(jnp.int32, (_OUT, _OUT), 1).astype(jnp.float32)
    ks = jax.lax.broadcasted_iota(jnp.int32, (_OUT, _OUT), 0)
    src_c = jnp.clip((j2 + 0.5) * (wc / _OUT) - 0.5, 0.0, wc - 1.0)
    c0f = jnp.floor(src_c)
    wcl = src_c - c0f
    c0 = c0f.astype(jnp.int32) + tly
    c1 = jnp.minimum(c0 + 1, bry - 1)
    cmat = jnp.where(ks == c0, 1.0 - wcl, 0.0) + jnp.where(ks == c1, wcl, 0.0)

    img = x_ref[0]
    mid = jnp.dot(img.reshape(3 * _OUT, _OUT), cmat,
                  preferred_element_type=jnp.float32)
    mid = mid.reshape(3, _OUT, _OUT)
    for ch in range(3):
        o_ref[0, 0, ch] = jnp.dot(rmat, mid[ch],
                                  preferred_element_type=jnp.float32)


def kernel(x, f):
    s, g = f.shape[0], f.shape[1]
    grid_spec = pltpu.PrefetchScalarGridSpec(
        num_scalar_prefetch=1,
        grid=(s, g),
        in_specs=[
            pl.BlockSpec((1, 3, _OUT, _OUT), lambda si, gi, fp: (si, 0, 0, 0)),
        ],
        out_specs=pl.BlockSpec((1, 1, 3, _OUT, _OUT),
                               lambda si, gi, fp: (si, gi, 0, 0, 0)),
    )
    return pl.pallas_call(
        _upsample_body,
        grid_spec=grid_spec,
        out_shape=jax.ShapeDtypeStruct((s, g, 3, _OUT, _OUT), jnp.float32),
        compiler_params=pltpu.CompilerParams(
            dimension_semantics=("parallel", "arbitrary"),
        ),
        name="roi_bilinear_upsample",
    )(f, x)


# speedup vs baseline: 4703.1848x; 1.1128x over previous
import jax
import jax.numpy as jnp
from jax.experimental import pallas as pl
from jax.experimental.pallas import tpu as pltpu

_OUT = 299


def _upsample_body(f_ref, x_ref, o_ref):
    si = pl.program_id(0)
    gi = pl.program_id(1)
    tlx = f_ref[si, gi, 0]
    tly = f_ref[si, gi, 1]
    brx = f_ref[si, gi, 2]
    bry = f_ref[si, gi, 3]
    hc = (brx - tlx).astype(jnp.float32)
    wc = (bry - tly).astype(jnp.float32)

    i2 = jax.lax.broadcasted_iota(jnp.int32, (_OUT, _OUT), 0).astype(jnp.float32)
    k2 = jax.lax.broadcasted_iota(jnp.int32, (_OUT, _OUT), 1)
    src_r = jnp.clip((i2 + 0.5) * (hc / _OUT) - 0.5, 0.0, hc - 1.0)
    r0f = jnp.floor(src_r)
    wr = src_r - r0f
    r0 = r0f.astype(jnp.int32) + tlx
    r1 = jnp.minimum(r0 + 1, brx - 1)
    rmat = jnp.where(k2 == r0, 1.0 - wr, 0.0) + jnp.where(k2 == r1, wr, 0.0)

    j2 = jax.lax.broadcasted_iota(jnp.int32, (_OUT, _OUT), 1).astype(jnp.float32)
    ks = jax.lax.broadcasted_iota(jnp.int32, (_OUT, _OUT), 0)
    src_c = jnp.clip((j2 + 0.5) * (wc / _OUT) - 0.5, 0.0, wc - 1.0)
    c0f = jnp.floor(src_c)
    wcl = src_c - c0f
    c0 = c0f.astype(jnp.int32) + tly
    c1 = jnp.minimum(c0 + 1, bry - 1)
    cmat = jnp.where(ks == c0, 1.0 - wcl, 0.0) + jnp.where(ks == c1, wcl, 0.0)

    for ch in range(3):
        mid = jnp.dot(x_ref[0, ch], cmat, preferred_element_type=jnp.float32)
        o_ref[0, 0, ch] = jnp.dot(rmat, mid, preferred_element_type=jnp.float32)


def kernel(x, f):
    s, g = f.shape[0], f.shape[1]
    grid_spec = pltpu.PrefetchScalarGridSpec(
        num_scalar_prefetch=1,
        grid=(s, g),
        in_specs=[
            pl.BlockSpec((1, 3, _OUT, _OUT), lambda si, gi, fp: (si, 0, 0, 0)),
        ],
        out_specs=pl.BlockSpec((1, 1, 3, _OUT, _OUT),
                               lambda si, gi, fp: (si, gi, 0, 0, 0)),
    )
    return pl.pallas_call(
        _upsample_body,
        grid_spec=grid_spec,
        out_shape=jax.ShapeDtypeStruct((s, g, 3, _OUT, _OUT), jnp.float32),
        compiler_params=pltpu.CompilerParams(
            dimension_semantics=("parallel", "arbitrary"),
        ),
        name="roi_bilinear_upsample",
    )(f, x)


# speedup vs baseline: 7318.1006x; 1.5560x over previous
import jax
import jax.numpy as jnp
from jax.experimental import pallas as pl
from jax.experimental.pallas import tpu as pltpu

_OUT = 299
_GBLK = 8


def _upsample_body(f_ref, x_ref, o_ref):
    si = pl.program_id(0)
    gi = pl.program_id(1)

    i2 = jax.lax.broadcasted_iota(jnp.int32, (_OUT, _OUT), 0).astype(jnp.float32)
    k2 = jax.lax.broadcasted_iota(jnp.int32, (_OUT, _OUT), 1)
    j2 = jax.lax.broadcasted_iota(jnp.int32, (_OUT, _OUT), 1).astype(jnp.float32)
    ks = jax.lax.broadcasted_iota(jnp.int32, (_OUT, _OUT), 0)

    for gsub in range(_GBLK):
        g = gi * _GBLK + gsub
        tlx = f_ref[si, g, 0]
        tly = f_ref[si, g, 1]
        brx = f_ref[si, g, 2]
        bry = f_ref[si, g, 3]
        hc = (brx - tlx).astype(jnp.float32)
        wc = (bry - tly).astype(jnp.float32)

        src_r = jnp.clip((i2 + 0.5) * (hc / _OUT) - 0.5, 0.0, hc - 1.0)
        r0f = jnp.floor(src_r)
        wr = src_r - r0f
        r0 = r0f.astype(jnp.int32) + tlx
        r1 = jnp.minimum(r0 + 1, brx - 1)
        rmat = jnp.where(k2 == r0, 1.0 - wr, 0.0) + jnp.where(k2 == r1, wr, 0.0)

        src_c = jnp.clip((j2 + 0.5) * (wc / _OUT) - 0.5, 0.0, wc - 1.0)
        c0f = jnp.floor(src_c)
        wcl = src_c - c0f
        c0 = c0f.astype(jnp.int32) + tly
        c1 = jnp.minimum(c0 + 1, bry - 1)
        cmat = jnp.where(ks == c0, 1.0 - wcl, 0.0) + jnp.where(ks == c1, wcl, 0.0)

        for ch in range(3):
            mid = jnp.dot(x_ref[0, ch], cmat, preferred_element_type=jnp.float32)
            o_ref[0, ch, :, gsub, :] = jnp.dot(rmat, mid,
                                               preferred_element_type=jnp.float32)


def kernel(x, f):
    s, g = f.shape[0], f.shape[1]
    grid_spec = pltpu.PrefetchScalarGridSpec(
        num_scalar_prefetch=1,
        grid=(s, g // _GBLK),
        in_specs=[
            pl.BlockSpec((1, 3, _OUT, _OUT),
                         lambda si, gi, fp: (si, 0, 0, 0)),
        ],
        out_specs=pl.BlockSpec((1, 3, _OUT, _GBLK, _OUT),
                               lambda si, gi, fp: (si, 0, 0, gi, 0)),
    )
    out5 = pl.pallas_call(
        _upsample_body,
        grid_spec=grid_spec,
        out_shape=jax.ShapeDtypeStruct((s, 3, _OUT, g, _OUT), jnp.float32),
        compiler_params=pltpu.CompilerParams(
            dimension_semantics=("parallel", "arbitrary"),
        ),
        name="roi_bilinear_upsample",
    )(f, x)
    return jnp.transpose(out5, (0, 3, 1, 2, 4))
